# baseline (speedup 1.0000x reference)
import jax
import jax.numpy as jnp
from jax.experimental import pallas as pl
from jax.experimental.pallas import tpu as pltpu


def _dma_body(H, B, idx_ref, ck, kk, cv, vv, ok, ov, sem_bulk, sem_tok):
    bulk = []
    for b in range(B):
        for src, dst in ((ck, ok), (cv, ov)):
            c = pltpu.make_async_copy(src.at[:, b], dst.at[b], sem_bulk)
            c.start()
            bulk.append(c)
    for c in bulk:
        c.wait()
    row0 = idx_ref[0] * H
    toks = []
    for b in range(B):
        for src, dst in ((kk, ok), (vv, ov)):
            c = pltpu.make_async_copy(src.at[b], dst.at[b, pl.ds(row0, H)],
                                      sem_tok)
            c.start()
            toks.append(c)
    for c in toks:
        c.wait()


def kernel(key, value, cached_ar_key, cached_ar_value, cache_ar_index):
    S, H, B, D = cached_ar_key.shape
    SH = S * H

    ck3 = cached_ar_key.reshape(SH, B, D)
    cv3 = cached_ar_value.reshape(SH, B, D)
    k3 = key.reshape(B, H, D)
    v3 = value.reshape(B, H, D)
    idx = jnp.asarray(cache_ar_index, jnp.int32).reshape(1)

    import functools
    body = functools.partial(_dma_body, H, B)

    any_spec = pl.BlockSpec(memory_space=pl.ANY)
    out_k, out_v = pl.pallas_call(
        body,
        in_specs=[
            pl.BlockSpec(memory_space=pltpu.SMEM),
            any_spec, any_spec, any_spec, any_spec,
        ],
        out_specs=[any_spec, any_spec],
        out_shape=[jax.ShapeDtypeStruct((B, SH, D), jnp.float32)] * 2,
        scratch_shapes=[pltpu.SemaphoreType.DMA, pltpu.SemaphoreType.DMA],
    )(idx, ck3, k3, cv3, v3)

    return out_k.reshape(B, S, H, D), out_v.reshape(B, S, H, D)


# speedup vs baseline: 20.9939x; 20.9939x over previous
import functools

import jax
import jax.numpy as jnp
from jax import lax
from jax.experimental import pallas as pl
from jax.experimental.pallas import tpu as pltpu
from jax.experimental.pallas import tpu_sc as plsc

_NBUF = 4
_CH = 128


def kernel(key, value, cached_ar_key, cached_ar_value, cache_ar_index):
    S, H, B, D = cached_ar_key.shape
    SH = S * H
    NC, NS = 2, 16
    SH2 = SH // NC
    nch = SH2 // _CH
    ngrp = nch // _NBUF
    assert SH2 % (_CH * _NBUF) == 0 and B == NS and H == 16

    ck3 = cached_ar_key.reshape(SH, B, D)
    cv3 = cached_ar_value.reshape(SH, B, D)
    k3 = key.reshape(B, H, D)
    v3 = value.reshape(B, H, D)
    idx_arr = jnp.full((16,), jnp.asarray(cache_ar_index, jnp.int32))

    mesh = plsc.VectorSubcoreMesh(core_axis_name="c", subcore_axis_name="s")

    @functools.partial(
        pl.kernel,
        mesh=mesh,
        out_type=[jax.ShapeDtypeStruct((B * SH, D), jnp.float32)] * 2,
        scratch_types=[
            pltpu.VMEM((_NBUF, _CH, D), jnp.float32),
            pltpu.VMEM((H, D), jnp.float32),
            pltpu.VMEM((16,), jnp.int32),
            pltpu.SMEM((16,), jnp.int32),
        ] + [pltpu.SemaphoreType.DMA] * (2 * _NBUF),
    )
    def sc_update(idx_hbm, k_hbm, v_hbm, ck_hbm, cv_hbm, ok_hbm, ov_hbm,
                  bufs, tbuf, idx_v, idx_s, *sems):
        semr = sems[:_NBUF]
        semw = sems[_NBUF:]
        c = lax.axis_index("c")
        s = lax.axis_index("s")
        b = (s % 8) * 2 + c
        half = s // 8
        base = half * SH2
        obase = b * SH + base

        del idx_s
        pltpu.sync_copy(idx_hbm, idx_v)
        t0 = idx_v[...][0] * H + b * SH

        for src, tok, dst in ((ck_hbm, k_hbm, ok_hbm),
                              (cv_hbm, v_hbm, ov_hbm)):
            def group(g, carry):
                for j in range(_NBUF):
                    i = g * _NBUF + j

                    @pl.when(g > 0)
                    def _():
                        pltpu.make_async_copy(
                            bufs.at[j], dst.at[pl.ds(obase, _CH)],
                            semw[j]).wait()

                    pltpu.async_copy(
                        src.at[pl.ds(base + i * _CH, _CH), b], bufs.at[j],
                        semr[j])
                for j in range(_NBUF):
                    i = g * _NBUF + j
                    pltpu.make_async_copy(
                        src.at[pl.ds(base, _CH), b], bufs.at[j],
                        semr[j]).wait()
                    pltpu.async_copy(
                        bufs.at[j], dst.at[pl.ds(obase + i * _CH, _CH)],
                        semw[j])
                return carry

            lax.fori_loop(0, ngrp, group, 0)
            for j in range(_NBUF):
                pltpu.make_async_copy(
                    bufs.at[j], dst.at[pl.ds(obase, _CH)], semw[j]).wait()

            plsc.subcore_barrier()

            @pl.when(half == 0)
            def _():
                pltpu.async_copy(tok.at[b], tbuf, semr[0]).wait()
                pltpu.async_copy(tbuf, dst.at[pl.ds(t0, H)], semr[0]).wait()

    out_k, out_v = sc_update(idx_arr, k3, v3, ck3, cv3)
    return out_k.reshape(B, S, H, D), out_v.reshape(B, S, H, D)
